# baseline (speedup 1.0000x reference)
import jax
import jax.numpy as jnp
from jax.experimental import pallas as pl
from jax.experimental.pallas import tpu as pltpu


_NUM_CHUNKS = 4


def _copy_kernel(src_ref, dst_ref, buf, in_sems, out_sems):
    rows = src_ref.shape[0]
    chunk = rows // _NUM_CHUNKS

    def in_copy(i):
        return pltpu.make_async_copy(
            src_ref.at[pl.ds(i * chunk, chunk)], buf.at[i], in_sems.at[i]
        )

    def out_copy(i):
        return pltpu.make_async_copy(
            buf.at[i], dst_ref.at[pl.ds(i * chunk, chunk)], out_sems.at[i]
        )

    ins = [in_copy(i) for i in range(_NUM_CHUNKS)]
    outs = [out_copy(i) for i in range(_NUM_CHUNKS)]
    ins[0].start()
    for i in range(_NUM_CHUNKS):
        ins[i].wait()
        if i + 1 < _NUM_CHUNKS:
            ins[i + 1].start()
        outs[i].start()
    for c in outs:
        c.wait()


def kernel(prototypes):
    rows, feat = prototypes.shape
    chunk = rows // _NUM_CHUNKS
    return pl.pallas_call(
        _copy_kernel,
        out_shape=jax.ShapeDtypeStruct(prototypes.shape, prototypes.dtype),
        in_specs=[pl.BlockSpec(memory_space=pl.ANY)],
        out_specs=pl.BlockSpec(memory_space=pl.ANY),
        scratch_shapes=[
            pltpu.VMEM((_NUM_CHUNKS, chunk, feat), prototypes.dtype),
            pltpu.SemaphoreType.DMA((_NUM_CHUNKS,)),
            pltpu.SemaphoreType.DMA((_NUM_CHUNKS,)),
        ],
    )(prototypes)


# speedup vs baseline: 1.5275x; 1.5275x over previous
import jax
import jax.numpy as jnp
from jax.experimental import pallas as pl
from jax.experimental.pallas import tpu as pltpu


_CHUNK_ROWS = (512, 1024, 2560, 4096)


def _copy_kernel(src_ref, dst_ref, buf, in_sems, out_sems):
    offs = [0]
    for r in _CHUNK_ROWS[:-1]:
        offs.append(offs[-1] + r)
    ins, outs = [], []
    for i, (o, r) in enumerate(zip(offs, _CHUNK_ROWS)):
        c = pltpu.make_async_copy(
            src_ref.at[pl.ds(o, r)], buf.at[pl.ds(o, r)], in_sems.at[i]
        )
        c.start()
        ins.append(c)
        outs.append(
            pltpu.make_async_copy(
                buf.at[pl.ds(o, r)], dst_ref.at[pl.ds(o, r)], out_sems.at[i]
            )
        )
    for i in range(len(_CHUNK_ROWS)):
        ins[i].wait()
        outs[i].start()
    for c in outs:
        c.wait()


def kernel(prototypes):
    rows, feat = prototypes.shape
    n = len(_CHUNK_ROWS)
    return pl.pallas_call(
        _copy_kernel,
        out_shape=jax.ShapeDtypeStruct(prototypes.shape, prototypes.dtype),
        in_specs=[pl.BlockSpec(memory_space=pl.ANY)],
        out_specs=pl.BlockSpec(memory_space=pl.ANY),
        scratch_shapes=[
            pltpu.VMEM((rows, feat), prototypes.dtype),
            pltpu.SemaphoreType.DMA((n,)),
            pltpu.SemaphoreType.DMA((n,)),
        ],
    )(prototypes)
